# baseline (speedup 1.0000x reference)
import functools

import jax
import jax.numpy as jnp
from jax.experimental import pallas as pl
from jax.experimental.pallas import tpu as pltpu

N = 10000
F = 512
BM = 200


def _xw_kernel(x_ref, w_ref, out_ref):
    out_ref[...] = jnp.dot(
        x_ref[...].astype(jnp.bfloat16),
        w_ref[...],
        preferred_element_type=jnp.float32,
    ).astype(jnp.bfloat16)


def _layer1_kernel(adj_ref, s_ref, w2_ref, b1_ref, out_ref):
    acc = jnp.dot(
        adj_ref[...].astype(jnp.bfloat16),
        s_ref[...],
        preferred_element_type=jnp.float32,
    )
    h = jnp.maximum(acc + b1_ref[...], 0.0)
    out_ref[...] = jnp.dot(
        h.astype(jnp.bfloat16),
        w2_ref[...],
        preferred_element_type=jnp.float32,
    ).astype(jnp.bfloat16)


def _layer2_kernel(adj_ref, hw_ref, b2_ref, out_ref):
    out_ref[...] = (
        jnp.dot(
            adj_ref[...].astype(jnp.bfloat16),
            hw_ref[...],
            preferred_element_type=jnp.float32,
        )
        + b2_ref[...]
    )


@jax.jit
def kernel(x, adj, W1, b1, W2, b2):
    grid = (N // BM,)
    params = pltpu.CompilerParams(dimension_semantics=("parallel",))

    s1 = pl.pallas_call(
        _xw_kernel,
        grid=grid,
        in_specs=[
            pl.BlockSpec((BM, F), lambda i: (i, 0)),
            pl.BlockSpec((F, F), lambda i: (0, 0)),
        ],
        out_specs=pl.BlockSpec((BM, F), lambda i: (i, 0)),
        out_shape=jax.ShapeDtypeStruct((N, F), jnp.bfloat16),
        compiler_params=params,
    )(x, W1.astype(jnp.bfloat16))

    hw = pl.pallas_call(
        _layer1_kernel,
        grid=grid,
        in_specs=[
            pl.BlockSpec((BM, N), lambda i: (i, 0)),
            pl.BlockSpec((N, F), lambda i: (0, 0)),
            pl.BlockSpec((F, F), lambda i: (0, 0)),
            pl.BlockSpec((1, F), lambda i: (0, 0)),
        ],
        out_specs=pl.BlockSpec((BM, F), lambda i: (i, 0)),
        out_shape=jax.ShapeDtypeStruct((N, F), jnp.bfloat16),
        compiler_params=params,
    )(adj, s1, W2.astype(jnp.bfloat16), b1.reshape(1, F))

    out = pl.pallas_call(
        _layer2_kernel,
        grid=grid,
        in_specs=[
            pl.BlockSpec((BM, N), lambda i: (i, 0)),
            pl.BlockSpec((N, F), lambda i: (0, 0)),
            pl.BlockSpec((1, F), lambda i: (0, 0)),
        ],
        out_specs=pl.BlockSpec((BM, F), lambda i: (i, 0)),
        out_shape=jax.ShapeDtypeStruct((N, F), jnp.float32),
        compiler_params=params,
    )(adj, hw, b2.reshape(1, F))

    return out


# speedup vs baseline: 1.2066x; 1.2066x over previous
import functools

import jax
import jax.numpy as jnp
from jax.experimental import pallas as pl
from jax.experimental.pallas import tpu as pltpu

N = 10000
F = 512
BM = 400
NB = N // BM


def _xw_kernel(x_ref, w_ref, out_ref):
    out_ref[...] = jnp.dot(
        x_ref[...].astype(jnp.bfloat16),
        w_ref[...],
        preferred_element_type=jnp.float32,
    ).astype(jnp.bfloat16)


def _gcn_kernel(adj_ref, s_ref, w2_ref, b1_ref, b2_ref, out_ref, hw_ref):
    i = pl.program_id(0)

    @pl.when(i < NB)
    def _phase_a():
        acc = jnp.dot(
            adj_ref[...].astype(jnp.bfloat16),
            s_ref[...],
            preferred_element_type=jnp.float32,
        )
        h = jnp.maximum(acc + b1_ref[...], 0.0)
        hw_ref[pl.ds(i * BM, BM), :] = jnp.dot(
            h.astype(jnp.bfloat16),
            w2_ref[...],
            preferred_element_type=jnp.float32,
        ).astype(jnp.bfloat16)

    @pl.when(i >= NB)
    def _phase_b():
        out_ref[...] = (
            jnp.dot(
                adj_ref[...].astype(jnp.bfloat16),
                hw_ref[...],
                preferred_element_type=jnp.float32,
            )
            + b2_ref[...]
        )


@jax.jit
def kernel(x, adj, W1, b1, W2, b2):
    s1 = pl.pallas_call(
        _xw_kernel,
        grid=(NB,),
        in_specs=[
            pl.BlockSpec((BM, F), lambda i: (i, 0)),
            pl.BlockSpec((F, F), lambda i: (0, 0)),
        ],
        out_specs=pl.BlockSpec((BM, F), lambda i: (i, 0)),
        out_shape=jax.ShapeDtypeStruct((N, F), jnp.bfloat16),
        compiler_params=pltpu.CompilerParams(dimension_semantics=("parallel",)),
    )(x, W1.astype(jnp.bfloat16))

    out = pl.pallas_call(
        _gcn_kernel,
        grid=(2 * NB,),
        in_specs=[
            pl.BlockSpec((BM, N), lambda i: (jnp.where(i < NB, i, i - NB), 0)),
            pl.BlockSpec((N, F), lambda i: (0, 0)),
            pl.BlockSpec((F, F), lambda i: (0, 0)),
            pl.BlockSpec((1, F), lambda i: (0, 0)),
            pl.BlockSpec((1, F), lambda i: (0, 0)),
        ],
        out_specs=pl.BlockSpec(
            (BM, F), lambda i: (jnp.where(i < NB, 0, i - NB), 0)
        ),
        out_shape=jax.ShapeDtypeStruct((N, F), jnp.float32),
        scratch_shapes=[pltpu.VMEM((N, F), jnp.bfloat16)],
        compiler_params=pltpu.CompilerParams(dimension_semantics=("arbitrary",)),
    )(adj, s1, W2.astype(jnp.bfloat16), b1.reshape(1, F), b2.reshape(1, F))

    return out


# speedup vs baseline: 1.2309x; 1.0201x over previous
import functools

import jax
import jax.numpy as jnp
from jax.experimental import pallas as pl
from jax.experimental.pallas import tpu as pltpu

N = 10000
F = 512
BM = 400
NB = N // BM


def _gcn_kernel(
    x_ref, adj_ref, w1_ref, w2_ref, b1_ref, b2_ref, out_ref, s1_ref, hw_ref
):
    i = pl.program_id(0)

    @pl.when(i < NB)
    def _phase_0():
        s1_ref[pl.ds(i * BM, BM), :] = jnp.dot(
            x_ref[...].astype(jnp.bfloat16),
            w1_ref[...],
            preferred_element_type=jnp.float32,
        ).astype(jnp.bfloat16)

    @pl.when(jnp.logical_and(i >= NB, i < 2 * NB))
    def _phase_a():
        j = i - NB
        acc = jnp.dot(
            adj_ref[...].astype(jnp.bfloat16),
            s1_ref[...],
            preferred_element_type=jnp.float32,
        )
        h = jnp.maximum(acc + b1_ref[...], 0.0)
        hw_ref[pl.ds(j * BM, BM), :] = jnp.dot(
            h.astype(jnp.bfloat16),
            w2_ref[...],
            preferred_element_type=jnp.float32,
        ).astype(jnp.bfloat16)

    @pl.when(i >= 2 * NB)
    def _phase_b():
        out_ref[...] = (
            jnp.dot(
                adj_ref[...].astype(jnp.bfloat16),
                hw_ref[...],
                preferred_element_type=jnp.float32,
            )
            + b2_ref[...]
        )


@jax.jit
def kernel(x, adj, W1, b1, W2, b2):
    out = pl.pallas_call(
        _gcn_kernel,
        grid=(3 * NB,),
        in_specs=[
            pl.BlockSpec((BM, F), lambda i: (jnp.where(i < NB, i, 0), 0)),
            pl.BlockSpec(
                (BM, N),
                lambda i: (
                    jnp.where(i < NB, 0, jnp.where(i < 2 * NB, i - NB, i - 2 * NB)),
                    0,
                ),
            ),
            pl.BlockSpec((F, F), lambda i: (0, 0)),
            pl.BlockSpec((F, F), lambda i: (0, 0)),
            pl.BlockSpec((1, F), lambda i: (0, 0)),
            pl.BlockSpec((1, F), lambda i: (0, 0)),
        ],
        out_specs=pl.BlockSpec(
            (BM, F), lambda i: (jnp.where(i < 2 * NB, 0, i - 2 * NB), 0)
        ),
        out_shape=jax.ShapeDtypeStruct((N, F), jnp.float32),
        scratch_shapes=[
            pltpu.VMEM((N, F), jnp.bfloat16),
            pltpu.VMEM((N, F), jnp.bfloat16),
        ],
        compiler_params=pltpu.CompilerParams(
            dimension_semantics=("arbitrary",),
            vmem_limit_bytes=62 * 1024 * 1024,
        ),
    )(
        x,
        adj,
        W1.astype(jnp.bfloat16),
        W2.astype(jnp.bfloat16),
        b1.reshape(1, F),
        b2.reshape(1, F),
    )

    return out


# speedup vs baseline: 1.2701x; 1.0318x over previous
import functools

import jax
import jax.numpy as jnp
from jax.experimental import pallas as pl
from jax.experimental.pallas import tpu as pltpu

N = 10000
F = 512
BM = 400
NB = N // BM
BX = 1000
NX = N // BX


def _gcn_kernel(
    x_ref, adj_ref, w1_ref, w2_ref, b1_ref, b2_ref, out_ref, s1_ref, hw_ref
):
    i = pl.program_id(0)

    @pl.when(i < NX)
    def _phase_0():
        s1_ref[pl.ds(i * BX, BX), :] = jnp.dot(
            x_ref[...].astype(jnp.bfloat16),
            w1_ref[...],
            preferred_element_type=jnp.float32,
        ).astype(jnp.bfloat16)

    @pl.when(jnp.logical_and(i >= NX, i < NX + NB))
    def _phase_a():
        j = i - NX
        acc = jnp.dot(
            adj_ref[...].astype(jnp.bfloat16),
            s1_ref[...],
            preferred_element_type=jnp.float32,
        )
        h = jnp.maximum(acc + b1_ref[...], 0.0)
        hw_ref[pl.ds(j * BM, BM), :] = jnp.dot(
            h.astype(jnp.bfloat16),
            w2_ref[...],
            preferred_element_type=jnp.float32,
        ).astype(jnp.bfloat16)

    @pl.when(i >= NX + NB)
    def _phase_b():
        out_ref[...] = (
            jnp.dot(
                adj_ref[...].astype(jnp.bfloat16),
                hw_ref[...],
                preferred_element_type=jnp.float32,
            )
            + b2_ref[...]
        )


@jax.jit
def kernel(x, adj, W1, b1, W2, b2):
    out = pl.pallas_call(
        _gcn_kernel,
        grid=(NX + 2 * NB,),
        in_specs=[
            pl.BlockSpec((BX, F), lambda i: (jnp.where(i < NX, i, 0), 0)),
            pl.BlockSpec(
                (BM, N),
                lambda i: (
                    jnp.where(
                        i < NX, 0, jnp.where(i < NX + NB, i - NX, i - NX - NB)
                    ),
                    0,
                ),
            ),
            pl.BlockSpec((F, F), lambda i: (0, 0)),
            pl.BlockSpec((F, F), lambda i: (0, 0)),
            pl.BlockSpec((1, F), lambda i: (0, 0)),
            pl.BlockSpec((1, F), lambda i: (0, 0)),
        ],
        out_specs=pl.BlockSpec(
            (BM, F), lambda i: (jnp.where(i < NX + NB, 0, i - NX - NB), 0)
        ),
        out_shape=jax.ShapeDtypeStruct((N, F), jnp.float32),
        scratch_shapes=[
            pltpu.VMEM((N, F), jnp.bfloat16),
            pltpu.VMEM((N, F), jnp.bfloat16),
        ],
        compiler_params=pltpu.CompilerParams(
            dimension_semantics=("arbitrary",),
            vmem_limit_bytes=62 * 1024 * 1024,
        ),
    )(
        x,
        adj,
        W1.astype(jnp.bfloat16),
        W2.astype(jnp.bfloat16),
        b1.reshape(1, F),
        b2.reshape(1, F),
    )

    return out


# speedup vs baseline: 1.2780x; 1.0063x over previous
import functools

import jax
import jax.numpy as jnp
from jax.experimental import pallas as pl
from jax.experimental.pallas import tpu as pltpu

N = 10000
F = 512
BM = 400
NB = N // BM
BX = 2000
NX = N // BX


def _gcn_kernel(
    x_ref, adj_ref, w1_ref, w2_ref, b1_ref, b2_ref, out_ref, s1_ref, hw_ref
):
    i = pl.program_id(0)

    @pl.when(i < NX)
    def _phase_0():
        s1_ref[pl.ds(i * BX, BX), :] = jnp.dot(
            x_ref[...].astype(jnp.bfloat16),
            w1_ref[...],
            preferred_element_type=jnp.float32,
        ).astype(jnp.bfloat16)

    @pl.when(jnp.logical_and(i >= NX, i < NX + NB))
    def _phase_a():
        j = i - NX
        acc = jnp.dot(
            adj_ref[...].astype(jnp.bfloat16),
            s1_ref[...],
            preferred_element_type=jnp.float32,
        )
        h = jnp.maximum(acc + b1_ref[...], 0.0)
        hw_ref[pl.ds(j * BM, BM), :] = jnp.dot(
            h.astype(jnp.bfloat16),
            w2_ref[...],
            preferred_element_type=jnp.float32,
        ).astype(jnp.bfloat16)

    @pl.when(i >= NX + NB)
    def _phase_b():
        out_ref[...] = (
            jnp.dot(
                adj_ref[...].astype(jnp.bfloat16),
                hw_ref[...],
                preferred_element_type=jnp.float32,
            )
            + b2_ref[...]
        )


@jax.jit
def kernel(x, adj, W1, b1, W2, b2):
    out = pl.pallas_call(
        _gcn_kernel,
        grid=(NX + 2 * NB,),
        in_specs=[
            pl.BlockSpec((BX, F), lambda i: (jnp.where(i < NX, i, 0), 0)),
            pl.BlockSpec(
                (BM, N),
                lambda i: (
                    jnp.where(
                        i < NX, 0, jnp.where(i < NX + NB, i - NX, i - NX - NB)
                    ),
                    0,
                ),
            ),
            pl.BlockSpec((F, F), lambda i: (0, 0)),
            pl.BlockSpec((F, F), lambda i: (0, 0)),
            pl.BlockSpec((1, F), lambda i: (0, 0)),
            pl.BlockSpec((1, F), lambda i: (0, 0)),
        ],
        out_specs=pl.BlockSpec(
            (BM, F), lambda i: (jnp.where(i < NX + NB, 0, i - NX - NB), 0)
        ),
        out_shape=jax.ShapeDtypeStruct((N, F), jnp.float32),
        scratch_shapes=[
            pltpu.VMEM((N, F), jnp.bfloat16),
            pltpu.VMEM((N, F), jnp.bfloat16),
        ],
        compiler_params=pltpu.CompilerParams(
            dimension_semantics=("arbitrary",),
            vmem_limit_bytes=63 * 1024 * 1024,
        ),
    )(
        x,
        adj,
        W1.astype(jnp.bfloat16),
        W2.astype(jnp.bfloat16),
        b1.reshape(1, F),
        b2.reshape(1, F),
    )

    return out
